# baseline (speedup 1.0000x reference)
import jax
import jax.numpy as jnp
from jax import lax
from jax.experimental import pallas as pl
from jax.experimental.pallas import tpu as pltpu
from jax.experimental.pallas import tpu_sc as plsc

N_NODES = 10000
N_EDGES = 320000
D = 128

NC = 2
NS = 16
NW = NC * NS
CHUNK = 128
CPW = 80
N_PAD = NW * CPW * CHUNK - N_EDGES

ROWS_PER_SUB = 624
TAIL_ROWS = N_NODES - NS * ROWS_PER_SUB
ZROWS = 104


def _matmul_body(x_ref, w_ref, o_ref):
    o_ref[...] = jnp.dot(x_ref[...], w_ref[...],
                         preferred_element_type=jnp.float32)


def _matmul(x, W):
    blk = 1000
    return pl.pallas_call(
        _matmul_body,
        grid=(N_NODES // blk,),
        in_specs=[
            pl.BlockSpec((blk, D), lambda i: (i, 0)),
            pl.BlockSpec((D, D), lambda i: (0, 0)),
        ],
        out_specs=pl.BlockSpec((blk, D), lambda i: (i, 0)),
        out_shape=jax.ShapeDtypeStruct((N_NODES, D), jnp.float32),
    )(x, W)


def _combine_body(p_ref, o_ref):
    o_ref[...] = jnp.maximum(p_ref[0] + p_ref[1], 0.0)


def _combine(partials):
    blk = 1000
    return pl.pallas_call(
        _combine_body,
        grid=(N_NODES // blk,),
        in_specs=[pl.BlockSpec((2, blk, D), lambda i: (0, i, 0))],
        out_specs=pl.BlockSpec((blk, D), lambda i: (i, 0)),
        out_shape=jax.ShapeDtypeStruct((N_NODES, D), jnp.float32),
    )(partials)


def _sc_body(support_hbm, src_hbm, dst_hbm, ew_hbm, out_hbm,
             acc_spmem,
             srcA, dstA, wA, rowsA, semA,
             srcB, dstB, wB, rowsB, semB):
    core = lax.axis_index("c")
    sub = lax.axis_index("s")
    wid = sub * NC + core

    @pl.loop(0, ZROWS)
    def _(r):
        for g in range(D // 16):
            rowsA[r, pl.ds(g * 16, 16)] = jnp.zeros((16,), jnp.float32)

    base = sub * ROWS_PER_SUB
    for k in range(ROWS_PER_SUB // ZROWS):
        pltpu.sync_copy(rowsA.at[pl.ds(0, ZROWS)],
                        acc_spmem.at[pl.ds(base + k * ZROWS, ZROWS)])

    @pl.when(sub == NS - 1)
    def _():
        pltpu.sync_copy(rowsA.at[pl.ds(0, TAIL_ROWS)],
                        acc_spmem.at[pl.ds(NS * ROWS_PER_SUB, TAIL_ROWS)])

    plsc.subcore_barrier()

    def load_and_gather(c, src_v, dst_v, w_v, rows, sem):
        e0 = c * CHUNK
        pltpu.sync_copy(src_hbm.at[pl.ds(e0, CHUNK)], src_v)
        pltpu.sync_copy(dst_hbm.at[pl.ds(e0, CHUNK)], dst_v)
        pltpu.sync_copy(ew_hbm.at[pl.ds(e0, CHUNK)], w_v)
        pltpu.async_copy(support_hbm.at[src_v], rows, sem)

    def finish(src_v, dst_v, w_v, rows, sem):
        pltpu.make_async_copy(support_hbm.at[src_v], rows, sem).wait()

        @pl.loop(0, CHUNK, step=16)
        def _(eg):
            w16 = w_v[pl.ds(eg, 16)]
            for j in range(16):
                bw = jnp.full((16,), w16[j], jnp.float32)
                for g in range(D // 16):
                    sl = pl.ds(g * 16, 16)
                    rows[eg + j, sl] = rows[eg + j, sl] * bw

        pltpu.sync_copy(rows, acc_spmem.at[dst_v], add=True)

    @pl.loop(0, CPW)
    def _(k):
        c = wid + k * NW
        e0 = c * CHUNK
        pltpu.sync_copy(src_hbm.at[pl.ds(e0, CHUNK)], srcA)
        pltpu.sync_copy(dst_hbm.at[pl.ds(e0, CHUNK)], dstA)
        pltpu.sync_copy(ew_hbm.at[pl.ds(e0, CHUNK)], wA)
        pltpu.async_copy(support_hbm.at[srcA], rowsA, semA).wait()

        @pl.loop(0, CHUNK, step=16)
        def _(eg):
            w16 = wA[pl.ds(eg, 16)]
            for j in range(16):
                bw = jnp.full((16,), w16[j], jnp.float32)
                for g in range(D // 16):
                    sl = pl.ds(g * 16, 16)
                    rowsA[eg + j, sl] = rowsA[eg + j, sl] * bw

        pltpu.sync_copy(rowsA, acc_spmem.at[dstA], add=True)

    plsc.subcore_barrier()

    ob = core * N_NODES + base
    for k in range(ROWS_PER_SUB // ZROWS):
        pltpu.sync_copy(acc_spmem.at[pl.ds(base + k * ZROWS, ZROWS)],
                        out_hbm.at[pl.ds(ob + k * ZROWS, ZROWS)])

    @pl.when(sub == NS - 1)
    def _():
        pltpu.sync_copy(acc_spmem.at[pl.ds(NS * ROWS_PER_SUB, TAIL_ROWS)],
                        out_hbm.at[pl.ds(core * N_NODES + NS * ROWS_PER_SUB,
                                         TAIL_ROWS)])


def _sc_spmm(support, src, dst, ew):
    mesh = plsc.VectorSubcoreMesh(core_axis_name="c", subcore_axis_name="s")
    f = pl.kernel(
        _sc_body,
        out_type=jax.ShapeDtypeStruct((NC * N_NODES, D), jnp.float32),
        mesh=mesh,
        scratch_types=[
            pltpu.VMEM_SHARED((N_NODES, D), jnp.float32),
            pltpu.VMEM((CHUNK,), jnp.int32),
            pltpu.VMEM((CHUNK,), jnp.int32),
            pltpu.VMEM((CHUNK,), jnp.float32),
            pltpu.VMEM((CHUNK, D), jnp.float32),
            pltpu.SemaphoreType.DMA,
            pltpu.VMEM((CHUNK,), jnp.int32),
            pltpu.VMEM((CHUNK,), jnp.int32),
            pltpu.VMEM((CHUNK,), jnp.float32),
            pltpu.VMEM((CHUNK, D), jnp.float32),
            pltpu.SemaphoreType.DMA,
        ],
    )
    return f(support, src, dst, ew)


def kernel(x, edge_index, edge_weight, W):
    support = _matmul(x, W)
    dst = jnp.pad(edge_index[0], (0, N_PAD))
    src = jnp.pad(edge_index[1], (0, N_PAD))
    ew = jnp.pad(edge_weight, (0, N_PAD))
    partials = _sc_spmm(support, src, dst, ew)
    return _combine(partials.reshape(NC, N_NODES, D))


# speedup vs baseline: 1.7059x; 1.7059x over previous
import jax
import jax.numpy as jnp
from jax import lax
from jax.experimental import pallas as pl
from jax.experimental.pallas import tpu as pltpu
from jax.experimental.pallas import tpu_sc as plsc

N_NODES = 10000
N_EDGES = 320000
D = 128

NC = 2
NS = 16
NW = NC * NS
CHUNK = 128
CPW = 80
N_PAD = NW * CPW * CHUNK - N_EDGES

ROWS_PER_SUB = 624
TAIL_ROWS = N_NODES - NS * ROWS_PER_SUB
ZROWS = 104


def _matmul_body(x_ref, w_ref, o_ref):
    o_ref[...] = jnp.dot(x_ref[...], w_ref[...],
                         preferred_element_type=jnp.float32)


def _matmul(x, W):
    blk = 1000
    return pl.pallas_call(
        _matmul_body,
        grid=(N_NODES // blk,),
        in_specs=[
            pl.BlockSpec((blk, D), lambda i: (i, 0)),
            pl.BlockSpec((D, D), lambda i: (0, 0)),
        ],
        out_specs=pl.BlockSpec((blk, D), lambda i: (i, 0)),
        out_shape=jax.ShapeDtypeStruct((N_NODES, D), jnp.float32),
    )(x, W)


def _combine_body(p_ref, o_ref):
    o_ref[...] = jnp.maximum(p_ref[0] + p_ref[1], 0.0)


def _combine(partials):
    blk = 1000
    return pl.pallas_call(
        _combine_body,
        grid=(N_NODES // blk,),
        in_specs=[pl.BlockSpec((2, blk, D), lambda i: (0, i, 0))],
        out_specs=pl.BlockSpec((blk, D), lambda i: (i, 0)),
        out_shape=jax.ShapeDtypeStruct((N_NODES, D), jnp.float32),
    )(partials)


def _sc_body(support_hbm, src_hbm, dst_hbm, ew_hbm, out_hbm,
             acc_spmem,
             srcA, dstA, wA, rowsA, semA,
             srcB, dstB, wB, rowsB, semB):
    core = lax.axis_index("c")
    sub = lax.axis_index("s")
    wid = sub * NC + core

    @pl.loop(0, ZROWS)
    def _(r):
        for g in range(D // 16):
            rowsA[r, pl.ds(g * 16, 16)] = jnp.zeros((16,), jnp.float32)

    base = sub * ROWS_PER_SUB
    for k in range(ROWS_PER_SUB // ZROWS):
        pltpu.sync_copy(rowsA.at[pl.ds(0, ZROWS)],
                        acc_spmem.at[pl.ds(base + k * ZROWS, ZROWS)])

    @pl.when(sub == NS - 1)
    def _():
        pltpu.sync_copy(rowsA.at[pl.ds(0, TAIL_ROWS)],
                        acc_spmem.at[pl.ds(NS * ROWS_PER_SUB, TAIL_ROWS)])

    plsc.subcore_barrier()

    def load_and_gather(c, src_v, dst_v, w_v, rows, sem):
        e0 = c * CHUNK
        pltpu.sync_copy(src_hbm.at[pl.ds(e0, CHUNK)], src_v)
        pltpu.sync_copy(dst_hbm.at[pl.ds(e0, CHUNK)], dst_v)
        pltpu.sync_copy(ew_hbm.at[pl.ds(e0, CHUNK)], w_v)
        pltpu.async_copy(support_hbm.at[src_v], rows, sem)

    def finish(src_v, dst_v, w_v, rows, sem):
        pltpu.make_async_copy(support_hbm.at[src_v], rows, sem).wait()

        @pl.loop(0, CHUNK, step=16)
        def _(eg):
            w16 = w_v[pl.ds(eg, 16)]
            for j in range(16):
                bw = jnp.full((16,), w16[j], jnp.float32)
                for g in range(D // 16):
                    sl = pl.ds(g * 16, 16)
                    rows[eg + j, sl] = rows[eg + j, sl] * bw

        pltpu.sync_copy(rows, acc_spmem.at[dst_v], add=True)

    @pl.loop(0, CPW)
    def _(k):
        c = wid + k * NW
        e0 = c * CHUNK
        pltpu.sync_copy(src_hbm.at[pl.ds(e0, CHUNK)], srcA)
        pltpu.sync_copy(dst_hbm.at[pl.ds(e0, CHUNK)], dstA)
        pltpu.sync_copy(ew_hbm.at[pl.ds(e0, CHUNK)], wA)
        pltpu.async_copy(support_hbm.at[srcA], rowsA, semA).wait()

        @pl.loop(0, CHUNK, step=16)
        def _(eg):
            w16 = wA[pl.ds(eg, 16)]
            for j in range(16):
                bw = jnp.full((16,), w16[j], jnp.float32)
                for g in range(D // 16):
                    sl = pl.ds(g * 16, 16)
                    rowsA[eg + j, sl] = rowsA[eg + j, sl] * bw

        pltpu.sync_copy(rowsA, acc_spmem.at[dstA], add=True)

    plsc.subcore_barrier()

    ob = core * N_NODES + base
    for k in range(ROWS_PER_SUB // ZROWS):
        pltpu.sync_copy(acc_spmem.at[pl.ds(base + k * ZROWS, ZROWS)],
                        out_hbm.at[pl.ds(ob + k * ZROWS, ZROWS)])

    @pl.when(sub == NS - 1)
    def _():
        pltpu.sync_copy(acc_spmem.at[pl.ds(NS * ROWS_PER_SUB, TAIL_ROWS)],
                        out_hbm.at[pl.ds(core * N_NODES + NS * ROWS_PER_SUB,
                                         TAIL_ROWS)])


def _sc_spmm(support, src, dst, ew):
    mesh = plsc.VectorSubcoreMesh(core_axis_name="c", subcore_axis_name="s")
    f = pl.kernel(
        _sc_body,
        out_type=jax.ShapeDtypeStruct((NC * N_NODES, D), jnp.float32),
        mesh=mesh,
        scratch_types=[
            pltpu.VMEM_SHARED((N_NODES, D), jnp.float32),
            pltpu.VMEM((CHUNK,), jnp.int32),
            pltpu.VMEM((CHUNK,), jnp.int32),
            pltpu.VMEM((CHUNK,), jnp.float32),
            pltpu.VMEM((CHUNK, D), jnp.float32),
            pltpu.SemaphoreType.DMA,
            pltpu.VMEM((CHUNK,), jnp.int32),
            pltpu.VMEM((CHUNK,), jnp.int32),
            pltpu.VMEM((CHUNK,), jnp.float32),
            pltpu.VMEM((CHUNK, D), jnp.float32),
            pltpu.SemaphoreType.DMA,
        ],
    )
    return f(support, src, dst, ew)


def kernel(x, edge_index, edge_weight, W):
    support = _matmul(x, W)
    pad_idx = jnp.arange(N_PAD, dtype=jnp.int32) % N_NODES
    dst = jnp.concatenate([edge_index[0], pad_idx])
    src = jnp.concatenate([edge_index[1], pad_idx])
    ew = jnp.pad(edge_weight, (0, N_PAD))
    partials = _sc_spmm(support, src, dst, ew)
    return _combine(partials.reshape(NC, N_NODES, D))


# speedup vs baseline: 2.3290x; 1.3653x over previous
import jax
import jax.numpy as jnp
from jax import lax
from jax.experimental import pallas as pl
from jax.experimental.pallas import tpu as pltpu
from jax.experimental.pallas import tpu_sc as plsc

N_NODES = 10000
N_EDGES = 320000
D = 128

NC = 2
NS = 16
NW = NC * NS
CHUNK = 128
CPW = 80
N_PAD = NW * CPW * CHUNK - N_EDGES

ROWS_PER_SUB = 624
TAIL_ROWS = N_NODES - NS * ROWS_PER_SUB
ZROWS = 104


def _matmul_body(x_ref, w_ref, o_ref):
    o_ref[...] = jnp.dot(x_ref[...], w_ref[...],
                         preferred_element_type=jnp.float32)


def _matmul(x, W):
    blk = 1000
    return pl.pallas_call(
        _matmul_body,
        grid=(N_NODES // blk,),
        in_specs=[
            pl.BlockSpec((blk, D), lambda i: (i, 0)),
            pl.BlockSpec((D, D), lambda i: (0, 0)),
        ],
        out_specs=pl.BlockSpec((blk, D), lambda i: (i, 0)),
        out_shape=jax.ShapeDtypeStruct((N_NODES, D), jnp.float32),
    )(x, W)


def _combine_body(p_ref, o_ref):
    o_ref[...] = jnp.maximum(p_ref[0] + p_ref[1], 0.0)


def _combine(partials):
    blk = 1000
    return pl.pallas_call(
        _combine_body,
        grid=(N_NODES // blk,),
        in_specs=[pl.BlockSpec((2, blk, D), lambda i: (0, i, 0))],
        out_specs=pl.BlockSpec((blk, D), lambda i: (i, 0)),
        out_shape=jax.ShapeDtypeStruct((N_NODES, D), jnp.float32),
    )(partials)


def _sc_body(support_hbm, src_hbm, dst_hbm, ew_hbm, out_hbm,
             acc_spmem,
             srcA, dstA, wA, rowsA, semA,
             srcB, dstB, wB, rowsB, semB):
    core = lax.axis_index("c")
    sub = lax.axis_index("s")
    wid = sub * NC + core

    @pl.loop(0, ZROWS)
    def _(r):
        for g in range(D // 16):
            rowsA[r, pl.ds(g * 16, 16)] = jnp.zeros((16,), jnp.float32)

    base = sub * ROWS_PER_SUB
    for k in range(ROWS_PER_SUB // ZROWS):
        pltpu.sync_copy(rowsA.at[pl.ds(0, ZROWS)],
                        acc_spmem.at[pl.ds(base + k * ZROWS, ZROWS)])

    @pl.when(sub == NS - 1)
    def _():
        pltpu.sync_copy(rowsA.at[pl.ds(0, TAIL_ROWS)],
                        acc_spmem.at[pl.ds(NS * ROWS_PER_SUB, TAIL_ROWS)])

    plsc.subcore_barrier()

    def load_and_gather(c, src_v, dst_v, w_v, rows, sem):
        e0 = c * CHUNK
        pltpu.sync_copy(src_hbm.at[pl.ds(e0, CHUNK)], src_v)
        pltpu.sync_copy(dst_hbm.at[pl.ds(e0, CHUNK)], dst_v)
        pltpu.sync_copy(ew_hbm.at[pl.ds(e0, CHUNK)], w_v)
        pltpu.async_copy(support_hbm.at[src_v], rows, sem)

    def finish(src_v, dst_v, w_v, rows, sem):
        pltpu.make_async_copy(support_hbm.at[src_v], rows, sem).wait()

        @pl.loop(0, CHUNK, step=16)
        def _(eg):
            w16 = w_v[pl.ds(eg, 16)]
            for j in range(16):
                bw = jnp.full((16,), w16[j], jnp.float32)
                for g in range(D // 16):
                    sl = pl.ds(g * 16, 16)
                    rows[eg + j, sl] = rows[eg + j, sl] * bw

        pltpu.sync_copy(rows, acc_spmem.at[dst_v], add=True)

    load_and_gather(wid, srcA, dstA, wA, rowsA, semA)
    load_and_gather(wid + NW, srcB, dstB, wB, rowsB, semB)

    @pl.loop(0, CPW, step=2)
    def _(k):
        c = wid + k * NW
        finish(srcA, dstA, wA, rowsA, semA)

        @pl.when(k + 2 < CPW)
        def _():
            load_and_gather(c + 2 * NW, srcA, dstA, wA, rowsA, semA)

        finish(srcB, dstB, wB, rowsB, semB)

        @pl.when(k + 3 < CPW)
        def _():
            load_and_gather(c + 3 * NW, srcB, dstB, wB, rowsB, semB)

    plsc.subcore_barrier()

    ob = core * N_NODES + base
    for k in range(ROWS_PER_SUB // ZROWS):
        pltpu.sync_copy(acc_spmem.at[pl.ds(base + k * ZROWS, ZROWS)],
                        out_hbm.at[pl.ds(ob + k * ZROWS, ZROWS)])

    @pl.when(sub == NS - 1)
    def _():
        pltpu.sync_copy(acc_spmem.at[pl.ds(NS * ROWS_PER_SUB, TAIL_ROWS)],
                        out_hbm.at[pl.ds(core * N_NODES + NS * ROWS_PER_SUB,
                                         TAIL_ROWS)])


def _sc_spmm(support, src, dst, ew):
    mesh = plsc.VectorSubcoreMesh(core_axis_name="c", subcore_axis_name="s")
    f = pl.kernel(
        _sc_body,
        out_type=jax.ShapeDtypeStruct((NC * N_NODES, D), jnp.float32),
        mesh=mesh,
        scratch_types=[
            pltpu.VMEM_SHARED((N_NODES, D), jnp.float32),
            pltpu.VMEM((CHUNK,), jnp.int32),
            pltpu.VMEM((CHUNK,), jnp.int32),
            pltpu.VMEM((CHUNK,), jnp.float32),
            pltpu.VMEM((CHUNK, D), jnp.float32),
            pltpu.SemaphoreType.DMA,
            pltpu.VMEM((CHUNK,), jnp.int32),
            pltpu.VMEM((CHUNK,), jnp.int32),
            pltpu.VMEM((CHUNK,), jnp.float32),
            pltpu.VMEM((CHUNK, D), jnp.float32),
            pltpu.SemaphoreType.DMA,
        ],
    )
    return f(support, src, dst, ew)


def kernel(x, edge_index, edge_weight, W):
    support = _matmul(x, W)
    pad_idx = jnp.arange(N_PAD, dtype=jnp.int32) % N_NODES
    dst = jnp.concatenate([edge_index[0], pad_idx])
    src = jnp.concatenate([edge_index[1], pad_idx])
    ew = jnp.pad(edge_weight, (0, N_PAD))
    partials = _sc_spmm(support, src, dst, ew)
    return _combine(partials.reshape(NC, N_NODES, D))


# speedup vs baseline: 2.3316x; 1.0011x over previous
import jax
import jax.numpy as jnp
from jax import lax
from jax.experimental import pallas as pl
from jax.experimental.pallas import tpu as pltpu
from jax.experimental.pallas import tpu_sc as plsc

N_NODES = 10000
N_EDGES = 320000
D = 128

NC = 2
NS = 16
NW = NC * NS
CHUNK = 128
CPW = 80
N_PAD = NW * CPW * CHUNK - N_EDGES

ROWS_PER_SUB = 624
TAIL_ROWS = N_NODES - NS * ROWS_PER_SUB
ZROWS = 104


def _matmul_body(x_ref, w_ref, o_ref):
    o_ref[...] = jnp.dot(x_ref[...], w_ref[...],
                         preferred_element_type=jnp.float32)


def _matmul(x, W):
    blk = 1000
    return pl.pallas_call(
        _matmul_body,
        grid=(N_NODES // blk,),
        in_specs=[
            pl.BlockSpec((blk, D), lambda i: (i, 0)),
            pl.BlockSpec((D, D), lambda i: (0, 0)),
        ],
        out_specs=pl.BlockSpec((blk, D), lambda i: (i, 0)),
        out_shape=jax.ShapeDtypeStruct((N_NODES, D), jnp.float32),
    )(x, W)


def _combine_body(p_ref, o_ref):
    o_ref[...] = jnp.maximum(p_ref[0] + p_ref[1], 0.0)


def _combine(partials):
    blk = 1000
    return pl.pallas_call(
        _combine_body,
        grid=(N_NODES // blk,),
        in_specs=[pl.BlockSpec((2, blk, D), lambda i: (0, i, 0))],
        out_specs=pl.BlockSpec((blk, D), lambda i: (i, 0)),
        out_shape=jax.ShapeDtypeStruct((N_NODES, D), jnp.float32),
    )(partials)


def _sc_body(support_hbm, src_hbm, dst_hbm, ew_hbm, out_hbm,
             acc_spmem,
             srcA, dstA, wA, rowsA, semA,
             srcB, dstB, wB, rowsB, semB):
    core = lax.axis_index("c")
    sub = lax.axis_index("s")
    wid = sub * NC + core

    @pl.loop(0, ZROWS)
    def _(r):
        for g in range(D // 16):
            rowsA[r, pl.ds(g * 16, 16)] = jnp.zeros((16,), jnp.float32)

    base = sub * ROWS_PER_SUB
    for k in range(ROWS_PER_SUB // ZROWS):
        pltpu.sync_copy(rowsA.at[pl.ds(0, ZROWS)],
                        acc_spmem.at[pl.ds(base + k * ZROWS, ZROWS)])

    @pl.when(sub == NS - 1)
    def _():
        pltpu.sync_copy(rowsA.at[pl.ds(0, TAIL_ROWS)],
                        acc_spmem.at[pl.ds(NS * ROWS_PER_SUB, TAIL_ROWS)])

    plsc.subcore_barrier()

    def load_and_gather(c, src_v, dst_v, w_v, rows, sem):
        e0 = c * CHUNK
        pltpu.sync_copy(src_hbm.at[pl.ds(e0, CHUNK)], src_v)
        pltpu.sync_copy(dst_hbm.at[pl.ds(e0, CHUNK)], dst_v)
        pltpu.sync_copy(ew_hbm.at[pl.ds(e0, CHUNK)], w_v)
        pltpu.async_copy(support_hbm.at[src_v], rows, sem)

    def finish(src_v, dst_v, w_v, rows, sem):
        pltpu.make_async_copy(support_hbm.at[src_v], rows, sem).wait()

        @plsc.parallel_loop(0, CHUNK, step=16, unroll=2)
        def _(eg):
            w16 = w_v[pl.ds(eg, 16)]
            for j in range(16):
                bw = jnp.full((16,), w16[j], jnp.float32)
                for g in range(D // 16):
                    sl = pl.ds(g * 16, 16)
                    rows[eg + j, sl] = rows[eg + j, sl] * bw

        pltpu.sync_copy(rows, acc_spmem.at[dst_v], add=True)

    load_and_gather(wid, srcA, dstA, wA, rowsA, semA)
    load_and_gather(wid + NW, srcB, dstB, wB, rowsB, semB)

    @pl.loop(0, CPW, step=2)
    def _(k):
        c = wid + k * NW
        finish(srcA, dstA, wA, rowsA, semA)

        @pl.when(k + 2 < CPW)
        def _():
            load_and_gather(c + 2 * NW, srcA, dstA, wA, rowsA, semA)

        finish(srcB, dstB, wB, rowsB, semB)

        @pl.when(k + 3 < CPW)
        def _():
            load_and_gather(c + 3 * NW, srcB, dstB, wB, rowsB, semB)

    plsc.subcore_barrier()

    ob = core * N_NODES + base
    for k in range(ROWS_PER_SUB // ZROWS):
        pltpu.sync_copy(acc_spmem.at[pl.ds(base + k * ZROWS, ZROWS)],
                        out_hbm.at[pl.ds(ob + k * ZROWS, ZROWS)])

    @pl.when(sub == NS - 1)
    def _():
        pltpu.sync_copy(acc_spmem.at[pl.ds(NS * ROWS_PER_SUB, TAIL_ROWS)],
                        out_hbm.at[pl.ds(core * N_NODES + NS * ROWS_PER_SUB,
                                         TAIL_ROWS)])


def _sc_spmm(support, src, dst, ew):
    mesh = plsc.VectorSubcoreMesh(core_axis_name="c", subcore_axis_name="s")
    f = pl.kernel(
        _sc_body,
        out_type=jax.ShapeDtypeStruct((NC * N_NODES, D), jnp.float32),
        mesh=mesh,
        scratch_types=[
            pltpu.VMEM_SHARED((N_NODES, D), jnp.float32),
            pltpu.VMEM((CHUNK,), jnp.int32),
            pltpu.VMEM((CHUNK,), jnp.int32),
            pltpu.VMEM((CHUNK,), jnp.float32),
            pltpu.VMEM((CHUNK, D), jnp.float32),
            pltpu.SemaphoreType.DMA,
            pltpu.VMEM((CHUNK,), jnp.int32),
            pltpu.VMEM((CHUNK,), jnp.int32),
            pltpu.VMEM((CHUNK,), jnp.float32),
            pltpu.VMEM((CHUNK, D), jnp.float32),
            pltpu.SemaphoreType.DMA,
        ],
    )
    return f(support, src, dst, ew)


def kernel(x, edge_index, edge_weight, W):
    support = _matmul(x, W)
    pad_idx = jnp.arange(N_PAD, dtype=jnp.int32) % N_NODES
    dst = jnp.concatenate([edge_index[0], pad_idx])
    src = jnp.concatenate([edge_index[1], pad_idx])
    ew = jnp.pad(edge_weight, (0, N_PAD))
    partials = _sc_spmm(support, src, dst, ew)
    return _combine(partials.reshape(NC, N_NODES, D))


# speedup vs baseline: 2.9962x; 1.2851x over previous
import dataclasses

import jax
import jax.numpy as jnp
from jax import lax
from jax.experimental import pallas as pl
from jax.experimental.pallas import tpu as pltpu
from jax.experimental.pallas import tpu_sc as plsc

N_NODES = 10000
N_EDGES = 320000
D = 128

NC = 2
NS = 16
NW = NC * NS
CHUNK = 128
CPW = 80
N_PAD = NW * CPW * CHUNK - N_EDGES

ROWS_PER_SUB = 624
TAIL_ROWS = N_NODES - NS * ROWS_PER_SUB
ZROWS = 104


def _matmul_body(x_ref, w_ref, o_ref):
    o_ref[...] = jnp.dot(x_ref[...], w_ref[...],
                         preferred_element_type=jnp.float32)


def _matmul(x, W):
    blk = 1000
    return pl.pallas_call(
        _matmul_body,
        grid=(N_NODES // blk,),
        in_specs=[
            pl.BlockSpec((blk, D), lambda i: (i, 0)),
            pl.BlockSpec((D, D), lambda i: (0, 0)),
        ],
        out_specs=pl.BlockSpec((blk, D), lambda i: (i, 0)),
        out_shape=jax.ShapeDtypeStruct((N_NODES, D), jnp.float32),
    )(x, W)


def _combine_body(p_ref, o_ref):
    o_ref[...] = jnp.maximum(p_ref[0] + p_ref[1], 0.0)


def _combine(partials):
    blk = 1000
    return pl.pallas_call(
        _combine_body,
        grid=(N_NODES // blk,),
        in_specs=[pl.BlockSpec((2, blk, D), lambda i: (0, i, 0))],
        out_specs=pl.BlockSpec((blk, D), lambda i: (i, 0)),
        out_shape=jax.ShapeDtypeStruct((N_NODES, D), jnp.float32),
    )(partials)


def _sc_body(support_hbm, meta_hbm, out_hbm,
             acc_spmem,
             metaA, rowsA, semA,
             metaB, rowsB, semB):
    core = lax.axis_index("c")
    sub = lax.axis_index("s")
    wid = sub * NC + core

    @pl.loop(0, ZROWS)
    def _(r):
        for g in range(D // 16):
            rowsA[r, pl.ds(g * 16, 16)] = jnp.zeros((16,), jnp.float32)

    base = sub * ROWS_PER_SUB
    for k in range(ROWS_PER_SUB // ZROWS):
        pltpu.sync_copy(rowsA.at[pl.ds(0, ZROWS)],
                        acc_spmem.at[pl.ds(base + k * ZROWS, ZROWS)])

    @pl.when(sub == NS - 1)
    def _():
        pltpu.sync_copy(rowsA.at[pl.ds(0, TAIL_ROWS)],
                        acc_spmem.at[pl.ds(NS * ROWS_PER_SUB, TAIL_ROWS)])

    plsc.subcore_barrier()

    def load_and_gather(c, meta_v, rows, sem):
        pltpu.sync_copy(meta_hbm.at[c], meta_v)
        pltpu.async_copy(support_hbm.at[meta_v.at[0]], rows, sem)

    def finish(meta_v, rows, sem):
        pltpu.make_async_copy(support_hbm.at[meta_v.at[0]], rows, sem).wait()

        @plsc.parallel_loop(0, CHUNK, step=16, unroll=2)
        def _(eg):
            w16 = plsc.bitcast(meta_v[2, pl.ds(eg, 16)], jnp.float32)
            for j in range(16):
                bw = jnp.full((16,), w16[j], jnp.float32)
                for g in range(D // 16):
                    sl = pl.ds(g * 16, 16)
                    rows[eg + j, sl] = rows[eg + j, sl] * bw

        pltpu.sync_copy(rows, acc_spmem.at[meta_v.at[1]], add=True)

    load_and_gather(wid, metaA, rowsA, semA)
    load_and_gather(wid + NW, metaB, rowsB, semB)

    @pl.loop(0, CPW, step=2)
    def _(k):
        c = wid + k * NW
        finish(metaA, rowsA, semA)

        @pl.when(k + 2 < CPW)
        def _():
            load_and_gather(c + 2 * NW, metaA, rowsA, semA)

        finish(metaB, rowsB, semB)

        @pl.when(k + 3 < CPW)
        def _():
            load_and_gather(c + 3 * NW, metaB, rowsB, semB)

    plsc.subcore_barrier()

    ob = core * N_NODES + base
    for k in range(ROWS_PER_SUB // ZROWS):
        pltpu.sync_copy(acc_spmem.at[pl.ds(base + k * ZROWS, ZROWS)],
                        out_hbm.at[pl.ds(ob + k * ZROWS, ZROWS)])

    @pl.when(sub == NS - 1)
    def _():
        pltpu.sync_copy(acc_spmem.at[pl.ds(NS * ROWS_PER_SUB, TAIL_ROWS)],
                        out_hbm.at[pl.ds(core * N_NODES + NS * ROWS_PER_SUB,
                                         TAIL_ROWS)])


def _sc_spmm(support, meta):
    mesh = plsc.VectorSubcoreMesh(core_axis_name="c", subcore_axis_name="s")
    cp = pltpu.CompilerParams()
    if "needs_layout_passes" in pltpu.CompilerParams.__dataclass_fields__:
        cp = dataclasses.replace(cp, needs_layout_passes=False)
    f = pl.kernel(
        _sc_body,
        compiler_params=cp,
        out_type=jax.ShapeDtypeStruct((NC * N_NODES, D), jnp.float32),
        mesh=mesh,
        scratch_types=[
            pltpu.VMEM_SHARED((N_NODES, D), jnp.float32),
            pltpu.VMEM((8, CHUNK), jnp.int32),
            pltpu.VMEM((CHUNK, D), jnp.float32),
            pltpu.SemaphoreType.DMA,
            pltpu.VMEM((8, CHUNK), jnp.int32),
            pltpu.VMEM((CHUNK, D), jnp.float32),
            pltpu.SemaphoreType.DMA,
        ],
    )
    return f(support, meta)


def kernel(x, edge_index, edge_weight, W):
    support = _matmul(x, W)
    pad_idx = jnp.arange(N_PAD, dtype=jnp.int32) % N_NODES
    dst = jnp.concatenate([edge_index[0], pad_idx]).reshape(-1, CHUNK)
    src = jnp.concatenate([edge_index[1], pad_idx]).reshape(-1, CHUNK)
    ew = jax.lax.bitcast_convert_type(
        jnp.pad(edge_weight, (0, N_PAD)), jnp.int32).reshape(-1, CHUNK)
    zpad = jnp.zeros((NW * CPW, 5, CHUNK), jnp.int32)
    meta = jnp.concatenate(
        [jnp.stack([src, dst, ew], axis=1), zpad], axis=1)
    partials = _sc_spmm(support, meta)
    return _combine(partials.reshape(NC, N_NODES, D))
